# Initial kernel scaffold; baseline (speedup 1.0000x reference)
import jax
import jax.numpy as jnp
from jax.experimental import pallas as pl


def kernel(x, W0, W1, W2, W3, W4, W5, W6, W7, W8):
    raise NotImplementedError("write your pallas kernel here")


# speedup vs baseline: 7.8439x; 7.8439x over previous
import jax
import jax.numpy as jnp
from jax.experimental import pallas as pl

EMB = 128
VOCABS = (119, 5, 12, 12, 10, 6, 6, 2, 2)
TOT = sum(VOCABS)
TOT_PAD = 176
BLOCK = 2048


def _body(x_ref, w_ref, o_ref):
    x = x_ref[...]
    w = w_ref[...]
    b = x.shape[0]
    iota = jax.lax.broadcasted_iota(jnp.int32, (b, TOT_PAD), 1)
    mh = jnp.zeros((b, TOT_PAD), jnp.float32)
    off = 0
    for i, v in enumerate(VOCABS):
        mh = mh + (iota == (x[:, i : i + 1] + off)).astype(jnp.float32)
        off += v
    o_ref[...] = jax.lax.dot_general(
        mh, w, (((1,), (0,)), ((), ())), preferred_element_type=jnp.float32
    )


def kernel(x, W0, W1, W2, W3, W4, W5, W6, W7, W8):
    n = x.shape[0]
    wcat = jnp.concatenate([W0, W1, W2, W3, W4, W5, W6, W7, W8], axis=0)
    wcat = jnp.pad(wcat, ((0, TOT_PAD - TOT), (0, 0)))
    n_pad = ((n + BLOCK - 1) // BLOCK) * BLOCK
    xp = jnp.pad(x, ((0, n_pad - n), (0, 0)))
    out = pl.pallas_call(
        _body,
        grid=(n_pad // BLOCK,),
        in_specs=[
            pl.BlockSpec((BLOCK, 9), lambda i: (i, 0)),
            pl.BlockSpec((TOT_PAD, EMB), lambda i: (0, 0)),
        ],
        out_specs=pl.BlockSpec((BLOCK, EMB), lambda i: (i, 0)),
        out_shape=jax.ShapeDtypeStruct((n_pad, EMB), jnp.float32),
    )(xp, wcat)
    return out[:n]
